# Initial kernel scaffold; baseline (speedup 1.0000x reference)
import jax
import jax.numpy as jnp
from jax.experimental import pallas as pl


def kernel(conv1_w, conv2_w, linear1_w, linear2_w, x):
    raise NotImplementedError("write your pallas kernel here")


# speedup vs baseline: 34.0987x; 34.0987x over previous
import jax
import jax.numpy as jnp
import numpy as np
from jax.experimental import pallas as pl
from jax.experimental.pallas import tpu as pltpu

_B = 64


def _pool_tap_mats(w, w_in):
    cout, cin, kh, kw = w.shape
    wp = (w_in - kw + 1) // 2
    xs = np.arange(w_in)[:, None, None]
    kj = np.arange(kw)[None, :, None]
    px = np.arange(wp)[None, None, :]
    d = xs - 2 * px - kj
    sel = jnp.asarray(((d == 0) | (d == 1)).astype(np.float32))
    t = jnp.einsum("xjp,oiyj->yxipo", sel, w)
    t = t.reshape(kh, w_in * cin, wp * cout)
    z = jnp.zeros_like(t[:1])
    tpad = jnp.concatenate([z, t, z], axis=0)
    return tpad[1:] + tpad[:-1]


def _lenet_block(xw_ref, u1e_ref, u1o_ref, u2_ref, w3_ref, w4_ref, o_ref):
    b = xw_ref.shape[2]
    xv = xw_ref[0].reshape(15 * b, 1536)

    a1e = jnp.dot(xv, u1e_ref[...], preferred_element_type=jnp.float32)
    a1o = jnp.dot(xv, u1o_ref[...], preferred_element_type=jnp.float32)
    p1e = jnp.maximum(a1e * 0.25, 0.0).astype(jnp.bfloat16).reshape(15, b, 512)
    p1o = jnp.maximum(a1o * 0.25, 0.0).astype(jnp.bfloat16).reshape(15, b, 512)

    ph = (p1e, p1o)
    fcat = jnp.concatenate(
        [ph[r % 2][r // 2:r // 2 + 13] for r in range(6)], axis=2)
    a2 = jnp.dot(fcat.reshape(13 * b, 3072), u2_ref[...],
                 preferred_element_type=jnp.float32)
    f = jnp.maximum(a2 * 0.25, 0.0).astype(jnp.bfloat16).reshape(13, b, 896)

    fl = jnp.concatenate([f[py] for py in range(13)], axis=1)
    h = jnp.dot(fl, w3_ref[...], preferred_element_type=jnp.float32)
    hb = jnp.maximum(h, 0.0).astype(jnp.bfloat16)

    o_ref[0] = jnp.dot(hb, w4_ref[...], preferred_element_type=jnp.float32)


def kernel(conv1_w, conv2_w, linear1_w, linear2_w, x):
    n, cin, hh, ww = x.shape
    c1 = conv1_w.shape[0]
    c2, _, kh, _ = conv2_w.shape
    hid, num_classes = linear1_w.shape[0], linear2_w.shape[0]
    wp1 = ((ww - kh + 1) // 2)
    wp2 = ((wp1 - kh + 1) // 2)
    hp2 = wp2
    k1, n1 = ww * cin, wp1 * c1
    k2, n2 = wp1 * c1, wp2 * c2
    n1p = 512
    n2p = 896
    nb = n // _B

    u1 = _pool_tap_mats(conv1_w, ww)
    u2 = _pool_tap_mats(conv2_w, wp1)
    u1p = jnp.pad(u1, ((0, 0), (0, 0), (0, n1p - n1)))
    u1e = jnp.zeros((8, k1, n1p), jnp.float32).at[0:kh + 1].set(u1p)
    u1o = jnp.zeros((8, k1, n1p), jnp.float32).at[2:kh + 3].set(u1p)
    u1e = u1e.reshape(8 * k1, n1p).astype(jnp.bfloat16)
    u1o = u1o.reshape(8 * k1, n1p).astype(jnp.bfloat16)
    u2p = jnp.pad(u2, ((0, 0), (0, n1p - k2), (0, n2p - n2)))
    u2b = u2p.reshape((kh + 1) * n1p, n2p).astype(jnp.bfloat16)
    w3k = linear1_w.reshape(hid, c2, hp2, wp2).transpose(2, 3, 1, 0)
    w3k = w3k.reshape(hp2, wp2 * c2, hid)
    w3b = jnp.pad(w3k, ((0, 0), (0, n2p - n2), (0, 0)))
    w3b = w3b.reshape(hp2 * n2p, hid).astype(jnp.bfloat16)
    w4b = linear2_w.T.astype(jnp.bfloat16)

    x_cl = jnp.transpose(x.astype(jnp.bfloat16), (0, 2, 3, 1))
    x_cl = x_cl.reshape(n, hh, k1)
    r4 = x_cl.reshape(n, hh // 4, 4 * k1)
    xw = jnp.concatenate([r4[:, 0:15, :], r4[:, 1:16, :]], axis=2)
    xw = xw.reshape(nb, _B, 15, 8 * k1).transpose(0, 2, 1, 3)

    out = pl.pallas_call(
        _lenet_block,
        out_shape=jax.ShapeDtypeStruct((nb, _B, num_classes), jnp.float32),
        grid=(nb,),
        in_specs=[
            pl.BlockSpec((1, 15, _B, 8 * k1), lambda i: (i, 0, 0, 0)),
            pl.BlockSpec((8 * k1, n1p), lambda i: (0, 0)),
            pl.BlockSpec((8 * k1, n1p), lambda i: (0, 0)),
            pl.BlockSpec(((kh + 1) * n1p, n2p), lambda i: (0, 0)),
            pl.BlockSpec((hp2 * n2p, hid), lambda i: (0, 0)),
            pl.BlockSpec((hid, num_classes), lambda i: (0, 0)),
        ],
        out_specs=pl.BlockSpec((1, _B, num_classes), lambda i: (i, 0, 0)),
        compiler_params=pltpu.CompilerParams(
            dimension_semantics=("parallel",)),
    )(xw, u1e, u1o, u2b, w3b, w4b)
    return out.reshape(n, num_classes)


# speedup vs baseline: 38.1163x; 1.1178x over previous
import jax
import jax.numpy as jnp
import numpy as np
from jax.experimental import pallas as pl
from jax.experimental.pallas import tpu as pltpu

_B = 64


def _pool_tap_mats(w, w_in):
    cout, cin, kh, kw = w.shape
    wp = (w_in - kw + 1) // 2
    xs = np.arange(w_in)[:, None, None]
    kj = np.arange(kw)[None, :, None]
    px = np.arange(wp)[None, None, :]
    d = xs - 2 * px - kj
    sel = jnp.asarray(((d == 0) | (d == 1)).astype(np.float32))
    t = jnp.einsum("xjp,oiyj->yxipo", sel, w)
    t = t.reshape(kh, w_in * cin, wp * cout)
    z = jnp.zeros_like(t[:1])
    tpad = jnp.concatenate([z, t, z], axis=0)
    return tpad[1:] + tpad[:-1]


def _lenet_block(x_ref, u1e_ref, u1o_ref, u2_ref, w3_ref, w4_ref, o_ref):
    b = x_ref.shape[3]
    xp = x_ref[0]
    xw = jnp.concatenate(
        [xp[w % 4, w // 4:w // 4 + 15] for w in range(8)], axis=2)
    xv = xw.reshape(15 * b, 2048)

    a1e = jnp.dot(xv, u1e_ref[...], preferred_element_type=jnp.float32)
    a1o = jnp.dot(xv, u1o_ref[...], preferred_element_type=jnp.float32)
    p1e = jnp.maximum(a1e * 0.25, 0.0).astype(jnp.bfloat16).reshape(15, b, 512)
    p1o = jnp.maximum(a1o * 0.25, 0.0).astype(jnp.bfloat16).reshape(15, b, 512)

    ph = (p1e, p1o)
    fcat = jnp.concatenate(
        [ph[r % 2][r // 2:r // 2 + 13] for r in range(6)], axis=2)
    a2 = jnp.dot(fcat.reshape(13 * b, 3072), u2_ref[...],
                 preferred_element_type=jnp.float32)
    f = jnp.maximum(a2 * 0.25, 0.0).astype(jnp.bfloat16).reshape(13, b, 896)

    fl = jnp.concatenate([f[py] for py in range(13)], axis=1)
    h = jnp.dot(fl, w3_ref[...], preferred_element_type=jnp.float32)
    hb = jnp.maximum(h, 0.0).astype(jnp.bfloat16)

    o_ref[0] = jnp.dot(hb, w4_ref[...], preferred_element_type=jnp.float32)


def kernel(conv1_w, conv2_w, linear1_w, linear2_w, x):
    n, cin, hh, ww = x.shape
    c1 = conv1_w.shape[0]
    c2, _, kh, _ = conv2_w.shape
    hid, num_classes = linear1_w.shape[0], linear2_w.shape[0]
    wp1 = ((ww - kh + 1) // 2)
    wp2 = ((wp1 - kh + 1) // 2)
    hp2 = wp2
    k1, n1 = ww * cin, wp1 * c1
    k2, n2 = wp1 * c1, wp2 * c2
    k1p = 256
    n1p = 512
    n2p = 896
    nb = n // _B

    u1 = _pool_tap_mats(conv1_w, ww)
    u2 = _pool_tap_mats(conv2_w, wp1)
    u1p = jnp.pad(u1, ((0, 0), (0, k1p - k1), (0, n1p - n1)))
    u1e = jnp.zeros((8, k1p, n1p), jnp.float32).at[0:kh + 1].set(u1p)
    u1o = jnp.zeros((8, k1p, n1p), jnp.float32).at[2:kh + 3].set(u1p)
    u1e = u1e.reshape(8 * k1p, n1p).astype(jnp.bfloat16)
    u1o = u1o.reshape(8 * k1p, n1p).astype(jnp.bfloat16)
    u2p = jnp.pad(u2, ((0, 0), (0, n1p - k2), (0, n2p - n2)))
    u2b = u2p.reshape((kh + 1) * n1p, n2p).astype(jnp.bfloat16)
    w3k = linear1_w.reshape(hid, c2, hp2, wp2).transpose(2, 3, 1, 0)
    w3k = w3k.reshape(hp2, wp2 * c2, hid)
    w3b = jnp.pad(w3k, ((0, 0), (0, n2p - n2), (0, 0)))
    w3b = w3b.reshape(hp2 * n2p, hid).astype(jnp.bfloat16)
    w4b = linear2_w.T.astype(jnp.bfloat16)

    xs = x.astype(jnp.bfloat16).reshape(nb, _B, cin, hh // 4, 4, ww)
    xs = xs.transpose(0, 4, 3, 1, 5, 2).reshape(nb, 4, hh // 4, _B, k1)
    xs = jnp.pad(xs, ((0, 0),) * 4 + ((0, k1p - k1),))

    out = pl.pallas_call(
        _lenet_block,
        out_shape=jax.ShapeDtypeStruct((nb, _B, num_classes), jnp.float32),
        grid=(nb,),
        in_specs=[
            pl.BlockSpec((1, 4, hh // 4, _B, k1p), lambda i: (i, 0, 0, 0, 0)),
            pl.BlockSpec((8 * k1p, n1p), lambda i: (0, 0)),
            pl.BlockSpec((8 * k1p, n1p), lambda i: (0, 0)),
            pl.BlockSpec(((kh + 1) * n1p, n2p), lambda i: (0, 0)),
            pl.BlockSpec((hp2 * n2p, hid), lambda i: (0, 0)),
            pl.BlockSpec((hid, num_classes), lambda i: (0, 0)),
        ],
        out_specs=pl.BlockSpec((1, _B, num_classes), lambda i: (i, 0, 0)),
        compiler_params=pltpu.CompilerParams(
            dimension_semantics=("parallel",)),
    )(xs, u1e, u1o, u2b, w3b, w4b)
    return out.reshape(n, num_classes)
